# baseline (speedup 1.0000x reference)
import dataclasses
import functools

import jax
import jax.numpy as jnp
from jax import lax
from jax.experimental import pallas as pl
from jax.experimental.pallas import tpu as pltpu
from jax.experimental.pallas import tpu_sc as plsc

B = 16384
V = 1000000
D = 64
K = 2
EPS = 1e-6

NC = 2
NS = 16
NW = NC * NS
BPW = B // NW
CHUNK = 128
NCH = BPW // CHUNK

GRP = 8
PACKW = GRP * 16
VP = V // GRP

TBL = 8192


def _tc_head_table(embT, wct):
    def body(embT_ref, wct_ref, o_ref):
        o_ref[...] = lax.dot_general(
            wct_ref[...], embT_ref[...],
            dimension_numbers=(((1,), (0,)), ((), ())),
            preferred_element_type=jnp.float32)

    return pl.pallas_call(
        body,
        grid=(pl.cdiv(V, TBL),),
        in_specs=[
            pl.BlockSpec((D, TBL), lambda i: (0, i)),
            pl.BlockSpec((16, D), lambda i: (0, 0)),
        ],
        out_specs=pl.BlockSpec((16, TBL), lambda i: (0, i)),
        out_shape=jax.ShapeDtypeStruct((16, V), jnp.float32),
    )(embT, wct)


def _sc_gather(pack, qidx2d):
    mesh = plsc.VectorSubcoreMesh(core_axis_name="c", subcore_axis_name="s")

    @functools.partial(
        pl.kernel,
        out_type=jax.ShapeDtypeStruct((B, PACKW), jnp.float32),
        mesh=mesh,
        scratch_types=[
            pltpu.VMEM((NCH, CHUNK), jnp.int32),
            pltpu.VMEM((BPW, PACKW), jnp.float32),
            pltpu.SemaphoreType.DMA,
        ],
    )
    def gather_kernel(pack_hbm, idx_hbm, out_hbm, idx_v, rows_v, sem):
        wid = lax.axis_index("s") * NC + lax.axis_index("c")
        pltpu.sync_copy(idx_hbm.at[wid], idx_v)
        copies = [
            pltpu.async_copy(
                pack_hbm.at[idx_v.at[c]],
                rows_v.at[pl.ds(c * CHUNK, CHUNK)], sem)
            for c in range(NCH)
        ]
        for cp in copies:
            cp.wait()
        pltpu.sync_copy(rows_v, out_hbm.at[pl.ds(wid * BPW, BPW)])

    return gather_kernel(pack, qidx2d)


def _tc_finish(gath, rem, u, b_t, wu_t):
    def body(g_ref, r_ref, u_ref, b_ref, wu_ref, o_ref):
        r = r_ref[...]
        raw = jnp.zeros((g_ref.shape[0], 16), jnp.float32)
        for j in range(GRP):
            raw = jnp.where(r == j, g_ref[:, j * 16:(j + 1) * 16], raw)
        raw = raw + b_ref[...] + u_ref[...] * wu_ref[...]
        lu = jax.nn.log_softmax(raw[:, 0:K], axis=-1)
        mu = raw[:, K:2 * K]
        su = jax.nn.softplus(raw[:, 2 * K:3 * K]) + EPS
        ld = jax.nn.log_softmax(raw[:, 6:6 + K], axis=-1)
        md = raw[:, 6 + K:6 + 2 * K]
        sd = jax.nn.softplus(raw[:, 6 + 2 * K:6 + 3 * K]) + EPS
        o_ref[...] = jnp.concatenate([lu, mu, su, ld, md, sd], axis=-1)

    tb = 2048
    return pl.pallas_call(
        body,
        grid=(B // tb,),
        in_specs=[
            pl.BlockSpec((tb, PACKW), lambda i: (i, 0)),
            pl.BlockSpec((tb, 1), lambda i: (i, 0)),
            pl.BlockSpec((tb, 1), lambda i: (i, 0)),
            pl.BlockSpec((1, 16), lambda i: (0, 0)),
            pl.BlockSpec((1, 16), lambda i: (0, 0)),
        ],
        out_specs=pl.BlockSpec((tb, 6 * K), lambda i: (i, 0)),
        out_shape=jax.ShapeDtypeStruct((B, 6 * K), jnp.float32),
    )(gath, rem, u, b_t, wu_t)


def kernel(source, upstream_speed, emb, W_up, b_up, W_down, b_down):
    src = source.astype(jnp.int32)
    w_cat = jnp.concatenate(
        [W_up, W_down[:D], jnp.zeros((D, 4), W_up.dtype)], axis=1)
    b_t = jnp.concatenate(
        [b_up, b_down, jnp.zeros((4,), b_up.dtype)])[None, :]
    wu_t = jnp.concatenate(
        [jnp.zeros((6,), W_down.dtype), W_down[D],
         jnp.zeros((4,), W_down.dtype)])[None, :]

    g_all = _tc_head_table(emb.T, w_cat.T)
    pack = g_all.T.reshape(VP, PACKW)
    qidx2d = (src // GRP).reshape(NW, NCH, CHUNK)
    gath = _sc_gather(pack, qidx2d)
    rem = (src % GRP)[:, None]
    u = upstream_speed[:, None]
    return _tc_finish(gath, rem, u, b_t, wu_t)


# speedup vs baseline: 2.5420x; 2.5420x over previous
import dataclasses
import functools

import jax
import jax.numpy as jnp
from jax import lax
from jax.experimental import pallas as pl
from jax.experimental.pallas import tpu as pltpu
from jax.experimental.pallas import tpu_sc as plsc

B = 16384
V = 1000000
D = 64
K = 2
EPS = 1e-6

NC = 2
NS = 16
NW = NC * NS
BPW = B // NW
CHUNK = 128
NCH = BPW // CHUNK


GRP = 8


def _sc_gather(table3, idx2d):
    mesh = plsc.VectorSubcoreMesh(core_axis_name="c", subcore_axis_name="s")
    cp = pltpu.CompilerParams()
    if "needs_layout_passes" in pltpu.CompilerParams.__dataclass_fields__:
        cp = dataclasses.replace(cp, needs_layout_passes=False)

    @functools.partial(
        pl.kernel,
        out_type=jax.ShapeDtypeStruct((B, D), jnp.float32),
        mesh=mesh,
        compiler_params=cp,
        scratch_types=[
            pltpu.VMEM((BPW,), jnp.int32),
            pltpu.VMEM((BPW, D), jnp.float32),
            pltpu.SemaphoreType.DMA,
            pltpu.SemaphoreType.DMA,
        ],
    )
    def gather_kernel(table_hbm, idx_hbm, out_hbm, idx_v, rows_v, isem, sem):
        wid = lax.axis_index("s") * NC + lax.axis_index("c")
        base = wid * BPW
        pltpu.async_copy(idx_hbm.at[wid], idx_v, isem).wait()
        lane = lax.broadcasted_iota(jnp.int32, (16,), 0)

        @pl.loop(0, BPW // 16)
        def _(g):
            v = idx_v[pl.ds(g * 16, 16)]
            for k in range(16):
                s = jnp.sum(jnp.where(lane == k, v, 0))
                pltpu.make_async_copy(
                    table_hbm.at[lax.shift_right_logical(s, 3), s & 7],
                    rows_v.at[g * 16 + k], sem).start()

        pltpu.make_async_copy(
            out_hbm.at[pl.ds(base, BPW)], rows_v, sem).wait()
        pltpu.sync_copy(rows_v, out_hbm.at[pl.ds(base, BPW)])

    return gather_kernel(table3, idx2d)


def _tc_heads(ctx, u, w_cat, b_cat, w_u):
    def body(ctx_ref, u_ref, wc_ref, bc_ref, wu_ref, o_ref):
        raw = jnp.dot(ctx_ref[...], wc_ref[...],
                      preferred_element_type=jnp.float32)
        raw = raw + bc_ref[...] + u_ref[...] * wu_ref[...]
        lu = jax.nn.log_softmax(raw[:, 0:K], axis=-1)
        mu = raw[:, K:2 * K]
        su = jax.nn.softplus(raw[:, 2 * K:3 * K]) + EPS
        ld = jax.nn.log_softmax(raw[:, 3 * K:4 * K], axis=-1)
        md = raw[:, 4 * K:5 * K]
        sd = jax.nn.softplus(raw[:, 5 * K:6 * K]) + EPS
        o_ref[...] = jnp.concatenate([lu, mu, su, ld, md, sd], axis=-1)

    tb = 2048
    return pl.pallas_call(
        body,
        grid=(B // tb,),
        in_specs=[
            pl.BlockSpec((tb, D), lambda i: (i, 0)),
            pl.BlockSpec((tb, 1), lambda i: (i, 0)),
            pl.BlockSpec((D, 6 * K), lambda i: (0, 0)),
            pl.BlockSpec((1, 6 * K), lambda i: (0, 0)),
            pl.BlockSpec((1, 6 * K), lambda i: (0, 0)),
        ],
        out_specs=pl.BlockSpec((tb, 6 * K), lambda i: (i, 0)),
        out_shape=jax.ShapeDtypeStruct((B, 6 * K), jnp.float32),
    )(ctx, u, w_cat, b_cat, w_u)


def kernel(source, upstream_speed, emb, W_up, b_up, W_down, b_down):
    src = source.astype(jnp.int32)
    idx2d = src.reshape(NW, BPW)
    table3 = emb.reshape(V // GRP, GRP, D)
    ctx = _sc_gather(table3, idx2d)
    w_cat = jnp.concatenate([W_up, W_down[:D]], axis=1)
    b_cat = jnp.concatenate([b_up, b_down])[None, :]
    w_u = jnp.concatenate(
        [jnp.zeros((3 * K,), W_down.dtype), W_down[D]])[None, :]
    u = upstream_speed[:, None]
    return _tc_heads(ctx, u, w_cat, b_cat, w_u)
